# Initial kernel scaffold; baseline (speedup 1.0000x reference)
import jax
import jax.numpy as jnp
from jax.experimental import pallas as pl


def kernel(x, edge_index_0, edge_weight_0, edge_index_1, edge_weight_1, edge_index_2, edge_weight_2, edge_index_3, edge_weight_3, W_0, W_1, W_2, W_3):
    raise NotImplementedError("write your pallas kernel here")


# speedup vs baseline: 6.1024x; 6.1024x over previous
import functools

import jax
import jax.numpy as jnp
from jax import lax
from jax.experimental import pallas as pl
from jax.experimental.pallas import tpu as pltpu
from jax.experimental.pallas import tpu_sc as plsc

_N = 10000
_E = 320000
_D = 128
_U = 128
_KH = 4

_NC = 2
_NS = 16
_NW = _NC * _NS
_G = 128
_GPH = _E // _G
_GPW = _GPH // _NW
_GX = _GPH - _GPW * _NW
_NP = 10240
_RPS = _NP // _NS


def _mm_body(x_ref, w_ref, o_ref):
    o_ref[0] = jnp.dot(x_ref[...], w_ref[0],
                       preferred_element_type=jnp.float32)


def _support_matmul(x, W):
    bn = 2000
    return pl.pallas_call(
        _mm_body,
        grid=(_KH, _N // bn),
        in_specs=[
            pl.BlockSpec((bn, _D), lambda k, i: (i, 0)),
            pl.BlockSpec((1, _D, _U), lambda k, i: (k, 0, 0)),
        ],
        out_specs=pl.BlockSpec((1, bn, _U), lambda k, i: (k, i, 0)),
        out_shape=jax.ShapeDtypeStruct((_KH, _N, _U), jnp.float32),
    )(x, W)


def _add_body(p_ref, o_ref):
    o_ref[...] = p_ref[0] + p_ref[1]


def _combine_partials(p):
    bn = 2000
    return pl.pallas_call(
        _add_body,
        grid=(_N // bn,),
        in_specs=[pl.BlockSpec((2, bn, _U), lambda i: (0, i, 0))],
        out_specs=pl.BlockSpec((bn, _U), lambda i: (i, 0)),
        out_shape=jax.ShapeDtypeStruct((_N, _U), jnp.float32),
    )(p)


def _sc_edge_kernel(support, eis, ews, zeros):
    mesh = plsc.VectorSubcoreMesh(core_axis_name="c", subcore_axis_name="s")

    @functools.partial(
        pl.kernel,
        out_type=jax.ShapeDtypeStruct((_NC, _NP, _U), jnp.float32),
        mesh=mesh,
        scratch_types=[
            pltpu.VMEM((_GPW, 1, _G), jnp.int32),
            pltpu.VMEM((_GPW, 1, _G), jnp.int32),
            pltpu.VMEM((_GPW, 1, _G), jnp.float32),
            pltpu.VMEM((1, 1, _G), jnp.int32),
            pltpu.VMEM((1, 1, _G), jnp.int32),
            pltpu.VMEM((1, 1, _G), jnp.float32),
            pltpu.VMEM((_G, _U), jnp.float32),
            pltpu.VMEM_SHARED((_NP, _U), jnp.float32),
            pltpu.SemaphoreType.DMA,
        ],
    )
    def k(sup_hbm, ei0, ei1, ei2, ei3, ew0, ew1, ew2, ew3, z_hbm, out_hbm,
          colb, rowb, ewb, colx, rowx, ewx, rows, acc, sem):
        cid = lax.axis_index("c")
        sid = lax.axis_index("s")
        wid = sid * _NC + cid

        pltpu.sync_copy(z_hbm.at[pl.ds(sid * _RPS, _RPS)],
                        acc.at[pl.ds(sid * _RPS, _RPS)])
        plsc.subcore_barrier()

        def process_group(cref, rref, wref, g):
            pltpu.async_copy(sup_hbm.at[cref.at[g, 0]], rows, sem).wait()

            def scale(b, _):
                base = b * 16
                wv = wref[g, 0, pl.ds(base, 16)]
                for i in range(16):
                    w = wv[i]
                    for j in range(_U // 16):
                        sl = pl.ds(j * 16, 16)
                        rows[base + i, sl] = rows[base + i, sl] * w
                return 0

            lax.fori_loop(0, _G // 16, scale, 0)

            pltpu.sync_copy(rows, acc.at[rref.at[g, 0]], add=True)

        for k_hop, (ei, ew) in enumerate(
                zip((ei0, ei1, ei2, ei3), (ew0, ew1, ew2, ew3))):
            gbase = wid * _GPW
            pltpu.sync_copy(ei.at[1, pl.ds(gbase, _GPW)], colb)
            pltpu.sync_copy(ei.at[0, pl.ds(gbase, _GPW)], rowb)
            pltpu.sync_copy(ew.at[pl.ds(gbase, _GPW)], ewb)

            off = jnp.int32(k_hop * _N)

            def adjust(i, _):
                sl = pl.ds(0, 16)
                r = i // (_G // 16)
                c = (i % (_G // 16)) * 16
                sl = pl.ds(c, 16)
                colb[r, 0, sl] = colb[r, 0, sl] + off
                return 0

            lax.fori_loop(0, _GPW * (_G // 16), adjust, 0, unroll=4)

            def group_body(g, _):
                process_group(colb, rowb, ewb, g)
                return 0

            lax.fori_loop(0, _GPW, group_body, 0)

            @pl.when(wid < _GX)
            def _():
                xg = _NW * _GPW + wid
                pltpu.sync_copy(ei.at[1, pl.ds(xg, 1)], colx)
                pltpu.sync_copy(ei.at[0, pl.ds(xg, 1)], rowx)
                pltpu.sync_copy(ew.at[pl.ds(xg, 1)], ewx)

                def adjx(i, _):
                    sl = pl.ds(i * 16, 16)
                    colx[0, 0, sl] = colx[0, 0, sl] + off
                    return 0

                lax.fori_loop(0, _G // 16, adjx, 0, unroll=4)
                process_group(colx, rowx, ewx, 0)

        plsc.subcore_barrier()
        pltpu.sync_copy(acc.at[pl.ds(sid * _RPS, _RPS)],
                        out_hbm.at[cid, pl.ds(sid * _RPS, _RPS)])

    return k(support, eis[0], eis[1], eis[2], eis[3],
             ews[0], ews[1], ews[2], ews[3], zeros)


def kernel(x, edge_index_0, edge_weight_0, edge_index_1, edge_weight_1,
           edge_index_2, edge_weight_2, edge_index_3, edge_weight_3,
           W_0, W_1, W_2, W_3):
    W = jnp.stack([W_0, W_1, W_2, W_3])
    support = _support_matmul(x, W).reshape(_KH * _N, _U)
    eis = [e.reshape(2, _GPH, 1, _G) for e in
           (edge_index_0, edge_index_1, edge_index_2, edge_index_3)]
    ews = [w.reshape(_GPH, 1, _G) for w in
           (edge_weight_0, edge_weight_1, edge_weight_2, edge_weight_3)]
    zeros = jnp.zeros((_NP, _U), jnp.float32)
    partials = _sc_edge_kernel(support, eis, ews, zeros)
    return _combine_partials(partials)


# speedup vs baseline: 8.0514x; 1.3194x over previous
import functools

import jax
import jax.numpy as jnp
from jax import lax
from jax.experimental import pallas as pl
from jax.experimental.pallas import tpu as pltpu
from jax.experimental.pallas import tpu_sc as plsc

_N = 10000
_E = 320000
_D = 128
_U = 128
_KH = 4

_NC = 2
_NS = 16
_NW = _NC * _NS
_G = 64
_GPW = 640
_SG = 40
_GPH = 5120
_EP = _GPH * _G
_NB = 4
_NP = 10240
_RPS = _NP // _NS


def _mm_body(x_ref, w_ref, o_ref):
    o_ref[0] = jnp.dot(x_ref[...], w_ref[0],
                       preferred_element_type=jnp.float32)


def _support_matmul(x, W):
    bn = 2000
    return pl.pallas_call(
        _mm_body,
        grid=(_KH, _N // bn),
        in_specs=[
            pl.BlockSpec((bn, _D), lambda k, i: (i, 0)),
            pl.BlockSpec((1, _D, _U), lambda k, i: (k, 0, 0)),
        ],
        out_specs=pl.BlockSpec((1, bn, _U), lambda k, i: (k, i, 0)),
        out_shape=jax.ShapeDtypeStruct((_KH, _N, _U), jnp.float32),
    )(x, W)


def _add_body(p_ref, o_ref):
    o_ref[...] = p_ref[0] + p_ref[1]


def _combine_partials(p):
    bn = 2000
    return pl.pallas_call(
        _add_body,
        grid=(_N // bn,),
        in_specs=[pl.BlockSpec((2, bn, _U), lambda i: (0, i, 0))],
        out_specs=pl.BlockSpec((bn, _U), lambda i: (i, 0)),
        out_shape=jax.ShapeDtypeStruct((_N, _U), jnp.float32),
    )(p)


def _sc_edge_kernel(support, col, row, ew, zeros):
    mesh = plsc.VectorSubcoreMesh(core_axis_name="c", subcore_axis_name="s")

    @functools.partial(
        pl.kernel,
        out_type=jax.ShapeDtypeStruct((_NC, _NP, _U), jnp.float32),
        mesh=mesh,
        scratch_types=[
            pltpu.VMEM((_SG, 1, _G), jnp.int32),
            pltpu.VMEM((_SG, 1, _G), jnp.int32),
            pltpu.VMEM((_SG, 1, _G), jnp.float32),
            [pltpu.VMEM((_G, _U), jnp.float32)] * _NB,
            [pltpu.SemaphoreType.DMA] * _NB,
            [pltpu.SemaphoreType.DMA] * _NB,
            pltpu.VMEM_SHARED((_NP, _U), jnp.float32),
        ],
    )
    def k(sup_hbm, col_hbm, row_hbm, ew_hbm, z_hbm, out_hbm,
          colb, rowb, ewb, rows, gsems, ssems, acc):
        cid = lax.axis_index("c")
        sid = lax.axis_index("s")
        wid = sid * _NC + cid

        pltpu.sync_copy(z_hbm.at[pl.ds(sid * _RPS, _RPS)],
                        acc.at[pl.ds(sid * _RPS, _RPS)])
        plsc.subcore_barrier()

        def gissue(g, b):
            pltpu.async_copy(sup_hbm.at[colb.at[g, 0]], rows[b], gsems[b])

        def gwait(b):
            pltpu.make_async_copy(z_hbm.at[pl.ds(0, _G)], rows[b],
                                  gsems[b]).wait()

        def sissue(g, b):
            pltpu.async_copy(rows[b], acc.at[rowb.at[g, 0]], ssems[b],
                             add=True)

        def swait(b):
            pltpu.make_async_copy(z_hbm.at[pl.ds(0, _G)], rows[b],
                                  ssems[b]).wait()

        def scale(g, b):
            def blk(bi, _):
                base = bi * 16
                wv = ewb[g, 0, pl.ds(base, 16)]
                for i in range(16):
                    w = wv[i]
                    for j in range(_U // 16):
                        sl = pl.ds(j * 16, 16)
                        rows[b][base + i, sl] = rows[b][base + i, sl] * w
                return 0

            lax.fori_loop(0, _G // 16, blk, 0)

        def chunk_body(c, _):
            gbase = wid * _GPW + c * _SG
            pltpu.sync_copy(col_hbm.at[pl.ds(gbase, _SG)], colb)
            pltpu.sync_copy(row_hbm.at[pl.ds(gbase, _SG)], rowb)
            pltpu.sync_copy(ew_hbm.at[pl.ds(gbase, _SG)], ewb)

            gissue(0, 0)
            gissue(1, 1)

            def step(t, _):
                for b in range(_NB):
                    g = t * _NB + b
                    b2 = (b + 2) % _NB

                    @pl.when(g + 2 < _SG)
                    def _():
                        @pl.when(g >= 2)
                        def _():
                            swait(b2)
                        gissue(g + 2, b2)

                    gwait(b)
                    scale(g, b)
                    sissue(g, b)
                return 0

            lax.fori_loop(0, _SG // _NB, step, 0)
            for b in range(_NB):
                swait(b)
            return 0

        lax.fori_loop(0, _GPW // _SG, chunk_body, 0)

        plsc.subcore_barrier()
        pltpu.sync_copy(acc.at[pl.ds(sid * _RPS, _RPS)],
                        out_hbm.at[cid, pl.ds(sid * _RPS, _RPS)])

    return k(support, col, row, ew, zeros)


def kernel(x, edge_index_0, edge_weight_0, edge_index_1, edge_weight_1,
           edge_index_2, edge_weight_2, edge_index_3, edge_weight_3,
           W_0, W_1, W_2, W_3):
    W = jnp.stack([W_0, W_1, W_2, W_3])
    support = _support_matmul(x, W).reshape(_KH * _N, _U)

    npad = _EP - _E
    pad_idx = jnp.arange(npad, dtype=jnp.int32) % _N
    pad_ew = jnp.zeros((npad,), jnp.float32)
    eis = (edge_index_0, edge_index_1, edge_index_2, edge_index_3)
    ews = (edge_weight_0, edge_weight_1, edge_weight_2, edge_weight_3)
    col = jnp.concatenate(
        [jnp.concatenate([e[1], pad_idx]) + jnp.int32(kh * _N)
         for kh, e in enumerate(eis)])
    row = jnp.concatenate(
        [jnp.concatenate([e[0], pad_idx]) for e in eis])
    eww = jnp.concatenate(
        [jnp.concatenate([w, pad_ew]) for w in ews])
    tg = _KH * _GPH
    col = col.reshape(tg, 1, _G)
    row = row.reshape(tg, 1, _G)
    eww = eww.reshape(tg, 1, _G)

    zeros = jnp.zeros((_NP, _U), jnp.float32)
    partials = _sc_edge_kernel(support, col, row, eww, zeros)
    return _combine_partials(partials)


# speedup vs baseline: 10.7646x; 1.3370x over previous
import functools

import jax
import jax.numpy as jnp
from jax import lax
from jax.experimental import pallas as pl
from jax.experimental.pallas import tpu as pltpu
from jax.experimental.pallas import tpu_sc as plsc

_N = 10000
_E = 320000
_D = 128
_U = 128
_KH = 4

_NC = 2
_NS = 16
_NW = _NC * _NS
_G = 64
_GPH = _E // _G
_GPW = 156
_SG = 40
_SGL = 36
_NCH = 4
_NB = 4
_NP = 10240
_RPS = _NP // _NS


def _mm_body(x_ref, w_ref, o_ref):
    o_ref[0] = jnp.dot(x_ref[...], w_ref[0],
                       preferred_element_type=jnp.float32)


def _support_matmul(x, W):
    bn = 2000
    return pl.pallas_call(
        _mm_body,
        grid=(_KH, _N // bn),
        in_specs=[
            pl.BlockSpec((bn, _D), lambda k, i: (i, 0)),
            pl.BlockSpec((1, _D, _U), lambda k, i: (k, 0, 0)),
        ],
        out_specs=pl.BlockSpec((1, bn, _U), lambda k, i: (k, i, 0)),
        out_shape=jax.ShapeDtypeStruct((_KH, _N, _U), jnp.float32),
    )(x, W)


def _add_body(p_ref, o_ref):
    o_ref[...] = p_ref[0] + p_ref[1]


def _combine_partials(p):
    bn = 2000
    return pl.pallas_call(
        _add_body,
        grid=(_N // bn,),
        in_specs=[pl.BlockSpec((2, bn, _U), lambda i: (0, i, 0))],
        out_specs=pl.BlockSpec((bn, _U), lambda i: (i, 0)),
        out_shape=jax.ShapeDtypeStruct((_N, _U), jnp.float32),
    )(p)


def _sc_edge_kernel(support, eis, ews, zeros):
    mesh = plsc.VectorSubcoreMesh(core_axis_name="c", subcore_axis_name="s")

    @functools.partial(
        pl.kernel,
        out_type=jax.ShapeDtypeStruct((_NC, _NP, _U), jnp.float32),
        mesh=mesh,
        scratch_types=[
            pltpu.VMEM((_SG, 1, _G), jnp.int32),
            pltpu.VMEM((_SG, 1, _G), jnp.int32),
            pltpu.VMEM((_SG, 1, _G), jnp.float32),
            pltpu.VMEM((1, 1, _G), jnp.int32),
            pltpu.VMEM((1, 1, _G), jnp.int32),
            pltpu.VMEM((1, 1, _G), jnp.float32),
            [pltpu.VMEM((_G, _U), jnp.float32)] * _NB,
            [pltpu.SemaphoreType.DMA] * _NB,
            [pltpu.SemaphoreType.DMA] * _NB,
            pltpu.VMEM_SHARED((_NP, _U), jnp.float32),
        ],
    )
    def k(sup_hbm, ei0, ei1, ei2, ei3, ew0, ew1, ew2, ew3, z_hbm, out_hbm,
          colb, rowb, ewb, colx, rowx, ewx, rows, gsems, ssems, acc):
        cid = lax.axis_index("c")
        sid = lax.axis_index("s")
        wid = sid * _NC + cid

        pltpu.sync_copy(z_hbm.at[pl.ds(sid * _RPS, _RPS)],
                        acc.at[pl.ds(sid * _RPS, _RPS)])
        plsc.subcore_barrier()

        def gissue(sup, cref, g, b):
            pltpu.async_copy(sup.at[cref.at[g, 0]], rows[b], gsems[b])

        def gwait(b):
            pltpu.make_async_copy(z_hbm.at[pl.ds(0, _G)], rows[b],
                                  gsems[b]).wait()

        def sissue(rref, g, b):
            pltpu.async_copy(rows[b], acc.at[rref.at[g, 0]], ssems[b],
                             add=True)

        def swait(b):
            pltpu.make_async_copy(z_hbm.at[pl.ds(0, _G)], rows[b],
                                  ssems[b]).wait()

        def scale(wref, g, b):
            def blk(bi, _):
                base = bi * 16
                wv = wref[g, 0, pl.ds(base, 16)]
                for i in range(16):
                    w = wv[i]
                    for j in range(_U // 16):
                        sl = pl.ds(j * 16, 16)
                        rows[b][base + i, sl] = rows[b][base + i, sl] * w
                return 0

            lax.fori_loop(0, _G // 16, blk, 0)

        for k_hop, (ei, ew) in enumerate(
                zip((ei0, ei1, ei2, ei3), (ew0, ew1, ew2, ew3))):
            sup = sup_hbm.at[pl.ds(k_hop * _N, _N)]

            def chunk_body(c, _):
                gbase = wid * _GPW + c * _SG
                pltpu.sync_copy(ei.at[1, pl.ds(gbase, _SG)], colb)
                pltpu.sync_copy(ei.at[0, pl.ds(gbase, _SG)], rowb)
                pltpu.sync_copy(ew.at[pl.ds(gbase, _SG)], ewb)
                n = lax.select(c < _NCH - 1, _SG, _SGL)

                gissue(sup, colb, 0, 0)
                gissue(sup, colb, 1, 1)

                def step(t, _):
                    for b in range(_NB):
                        g = t * _NB + b
                        b2 = (b + 2) % _NB

                        @pl.when(g + 2 < n)
                        def _():
                            @pl.when(g >= 2)
                            def _():
                                swait(b2)
                            gissue(sup, colb, g + 2, b2)

                        gwait(b)
                        scale(ewb, g, b)
                        sissue(rowb, g, b)
                    return 0

                lax.fori_loop(0, n // _NB, step, 0)
                for b in range(_NB):
                    swait(b)
                return 0

            lax.fori_loop(0, _NCH, chunk_body, 0)

            @pl.when(wid // 8 == k_hop)
            def _():
                xg = _NW * _GPW + (wid % 8)
                pltpu.sync_copy(ei.at[1, pl.ds(xg, 1)], colx)
                pltpu.sync_copy(ei.at[0, pl.ds(xg, 1)], rowx)
                pltpu.sync_copy(ew.at[pl.ds(xg, 1)], ewx)
                gissue(sup, colx, 0, 0)
                gwait(0)
                scale(ewx, 0, 0)
                sissue(rowx, 0, 0)
                swait(0)

        plsc.subcore_barrier()
        pltpu.sync_copy(acc.at[pl.ds(sid * _RPS, _RPS)],
                        out_hbm.at[cid, pl.ds(sid * _RPS, _RPS)])

    return k(support, eis[0], eis[1], eis[2], eis[3],
             ews[0], ews[1], ews[2], ews[3], zeros)


def kernel(x, edge_index_0, edge_weight_0, edge_index_1, edge_weight_1,
           edge_index_2, edge_weight_2, edge_index_3, edge_weight_3,
           W_0, W_1, W_2, W_3):
    W = jnp.stack([W_0, W_1, W_2, W_3])
    support = _support_matmul(x, W).reshape(_KH * _N, _U)
    eis = [e.reshape(2, _GPH, 1, _G) for e in
           (edge_index_0, edge_index_1, edge_index_2, edge_index_3)]
    ews = [w.reshape(_GPH, 1, _G) for w in
           (edge_weight_0, edge_weight_1, edge_weight_2, edge_weight_3)]
    zeros = jnp.zeros((_NP, _U), jnp.float32)
    partials = _sc_edge_kernel(support, eis, ews, zeros)
    return _combine_partials(partials)
